# Initial kernel scaffold; baseline (speedup 1.0000x reference)
import jax
import jax.numpy as jnp
from jax.experimental import pallas as pl


def kernel(boxes, anchors):
    raise NotImplementedError("write your pallas kernel here")


# speedup vs baseline: 624.9299x; 624.9299x over previous
import functools

import jax
import jax.numpy as jnp
from jax.experimental import pallas as pl
from jax.experimental.pallas import tpu as pltpu

_BPAD = 128
_BLKA = 2000
_VAR_XY = 0.1
_VAR_WH = 0.2
_THR = 0.5


def _iou_block(coords_t, anc):
    ax1, ay1, ax2, ay2 = (anc[:, k:k + 1] for k in range(4))
    bx1 = coords_t[0:1, :]
    by1 = coords_t[1:2, :]
    bx2 = coords_t[2:3, :]
    by2 = coords_t[3:4, :]
    iw = jnp.maximum(jnp.minimum(ax2, bx2) - jnp.maximum(ax1, bx1), 0.0)
    ih = jnp.maximum(jnp.minimum(ay2, by2) - jnp.maximum(ay1, by1), 0.0)
    inter = iw * ih
    area_a = (ax2 - ax1) * (ay2 - ay1)
    area_b = (bx2 - bx1) * (by2 - by1)
    return inter / (area_a + area_b - inter)


def _pass1_body(nblk, coords_t_ref, anc_ref, gmax_ref, gidx_ref, smax, sidx):
    i = pl.program_id(0)

    @pl.when(i == 0)
    def _init():
        smax[...] = jnp.full(smax.shape, -1.0, jnp.float32)
        sidx[...] = jnp.zeros(sidx.shape, jnp.int32)

    iou = _iou_block(coords_t_ref[...], anc_ref[...])
    bmax = jnp.max(iou, axis=0, keepdims=True)
    ridx = jax.lax.broadcasted_iota(jnp.int32, iou.shape, 0)
    bidx = jnp.min(jnp.where(iou == bmax, ridx, iou.shape[0]),
                   axis=0, keepdims=True) + i * _BLKA
    upd = bmax > smax[...]
    sidx[...] = jnp.where(upd, bidx, sidx[...])
    smax[...] = jnp.where(upd, bmax, smax[...])

    @pl.when(i == nblk - 1)
    def _emit():
        gmax_ref[...] = smax[...]
        gidx_ref[...] = sidx[...]


def _pass2_body(coords_t_ref, anc_ref, table_ref, gmax_ref, gidx_ref, out_ref):
    i = pl.program_id(0)
    anc = anc_ref[...]
    iou = _iou_block(coords_t_ref[...], anc)
    no_any = jnp.logical_not(gmax_ref[...] > _THR)
    aid = jax.lax.broadcasted_iota(jnp.int32, iou.shape, 0) + i * _BLKA
    amask = (iou > _THR) | (no_any & (aid == gidx_ref[...]))
    miou = jnp.where(amask, iou, 0.0)
    best = jnp.max(miou, axis=1, keepdims=True)
    lidx = jax.lax.broadcasted_iota(jnp.int32, miou.shape, 1)
    bidx = jnp.min(jnp.where(miou == best, lidx, _BPAD),
                   axis=1, keepdims=True)
    onehot = (lidx == bidx).astype(jnp.float32)
    sel = jnp.dot(onehot, table_ref[...],
                  preferred_element_type=jnp.float32,
                  precision=jax.lax.Precision.HIGHEST)

    ax1, ay1, ax2, ay2 = (anc[:, k:k + 1] for k in range(4))
    acx, acy = 0.5 * (ax1 + ax2), 0.5 * (ay1 + ay2)
    aw, ah = ax2 - ax1, ay2 - ay1
    bx1, by1, bx2, by2 = (sel[:, k:k + 1] for k in range(4))
    bcx, bcy = 0.5 * (bx1 + bx2), 0.5 * (by1 + by2)
    bw, bh = bx2 - bx1, by2 - by1

    pos = best > 0.0
    zero = jnp.zeros_like(best)
    ex = jnp.where(pos, (bcx - acx) / aw / _VAR_XY, zero)
    ey = jnp.where(pos, (bcy - acy) / ah / _VAR_XY, zero)
    ew = jnp.where(pos, jnp.log(jnp.maximum(bw, 1e-12) / aw) / _VAR_WH, zero)
    eh = jnp.where(pos, jnp.log(jnp.maximum(bh, 1e-12) / ah) / _VAR_WH, zero)
    mf = pos.astype(jnp.float32)
    cls = sel[:, 4:] * mf
    out_ref[...] = jnp.concatenate([ex, ey, ew, eh, 1.0 - mf, cls, mf], axis=1)


@jax.jit
def kernel(boxes, anchors):
    A = anchors.shape[0]
    B, C = boxes.shape
    nblk = A // _BLKA
    coords_t = jnp.zeros((4, _BPAD), jnp.float32).at[:, :B].set(boxes[:, :4].T)
    table = jnp.zeros((_BPAD, C), jnp.float32).at[:B, :].set(boxes)

    gmax, gidx = pl.pallas_call(
        functools.partial(_pass1_body, nblk),
        grid=(nblk,),
        in_specs=[
            pl.BlockSpec((4, _BPAD), lambda i: (0, 0)),
            pl.BlockSpec((_BLKA, 4), lambda i: (i, 0)),
        ],
        out_specs=[
            pl.BlockSpec((1, _BPAD), lambda i: (0, 0)),
            pl.BlockSpec((1, _BPAD), lambda i: (0, 0)),
        ],
        out_shape=[
            jax.ShapeDtypeStruct((1, _BPAD), jnp.float32),
            jax.ShapeDtypeStruct((1, _BPAD), jnp.int32),
        ],
        scratch_shapes=[
            pltpu.VMEM((1, _BPAD), jnp.float32),
            pltpu.VMEM((1, _BPAD), jnp.int32),
        ],
    )(coords_t, anchors)

    out = pl.pallas_call(
        _pass2_body,
        grid=(nblk,),
        in_specs=[
            pl.BlockSpec((4, _BPAD), lambda i: (0, 0)),
            pl.BlockSpec((_BLKA, 4), lambda i: (i, 0)),
            pl.BlockSpec((_BPAD, C), lambda i: (0, 0)),
            pl.BlockSpec((1, _BPAD), lambda i: (0, 0)),
            pl.BlockSpec((1, _BPAD), lambda i: (0, 0)),
        ],
        out_specs=pl.BlockSpec((_BLKA, C + 2), lambda i: (i, 0)),
        out_shape=jax.ShapeDtypeStruct((A, C + 2), jnp.float32),
    )(coords_t, anchors, table, gmax, gidx)
    return out
